# baseline (speedup 1.0000x reference)
import functools

import jax
import jax.numpy as jnp
from jax import lax
from jax.experimental import pallas as pl
from jax.experimental.pallas import tpu as pltpu
from jax.experimental.pallas import tpu_sc as plsc

N_NODES = 10000
N_EDGES = 320000
N_HEADS = 4
HEAD_DIM = 16
ROW = N_HEADS * HEAD_DIM

NUM_CORES = 2
NUM_SUBCORES = 16
NUM_WORKERS = NUM_CORES * NUM_SUBCORES
CHUNK = 200
NCHUNKS = 50
EDGES_PER_TILE = CHUNK * NCHUNKS
NBUF = 5
_UNROLL = 8
ROWS_PER_TILE = 640
N_PAD = NUM_SUBCORES * ROWS_PER_TILE

_IN_BOUNDS = jax.lax.GatherScatterMode.PROMISE_IN_BOUNDS


def _sc_body(v_hbm, srcp_hbm, dstp_hbm, wp_hbm, zeros_hbm, out_hbm,
             acc, src_v, dst_v, w_v, msg_v, *sems):
    gsems = sems[:NBUF]
    ssems = sems[NBUF:2 * NBUF]
    wsems = sems[2 * NBUF:]
    c = lax.axis_index("c")
    s = lax.axis_index("s")
    wid = c * NUM_SUBCORES + s

    pltpu.sync_copy(zeros_hbm, acc.at[pl.ds(s * ROWS_PER_TILE, ROWS_PER_TILE)])
    pltpu.sync_copy(srcp_hbm.at[wid], src_v)
    pltpu.sync_copy(dstp_hbm.at[wid], dst_v)
    plsc.subcore_barrier()

    wbase0 = wid * (EDGES_PER_TILE * N_HEADS)

    def fire_gather(k, b):
        pltpu.async_copy(v_hbm.at[src_v.at[k]], msg_v.at[b], gsems[b])
        pltpu.async_copy(wp_hbm.at[pl.ds(wbase0 + k * (CHUNK * N_HEADS),
                                         CHUNK * N_HEADS)],
                         w_v.at[b], wsems[b])

    def wait_gather(k, b):
        pltpu.make_async_copy(v_hbm.at[src_v.at[k]], msg_v.at[b], gsems[b]).wait()
        pltpu.make_async_copy(wp_hbm.at[pl.ds(wbase0 + k * (CHUNK * N_HEADS),
                                              CHUNK * N_HEADS)],
                              w_v.at[b], wsems[b]).wait()

    def fire_scatter(k, b):
        pltpu.async_copy(msg_v.at[b], acc.at[dst_v.at[k]], ssems[b], add=True)

    def wait_scatter(k, b):
        pltpu.make_async_copy(msg_v.at[b], acc.at[dst_v.at[k]], ssems[b]).wait()

    fire_gather(0, 0)
    fire_gather(1, 1)

    def group_body(g, carry):
        for p in range(NBUF):
            k = g * NBUF + p
            b2 = (p + 2) % NBUF
            @pl.when(k <= NCHUNKS - 3)
            def _():
                @pl.when(k >= 3)
                def _():
                    wait_scatter(k - 3, b2)
                fire_gather(k + 2, b2)

            wait_gather(k, p)
            mb = msg_v.at[p]
            wb = w_v.at[p]

            def e_body(eb, carry2):
                base = eb * _UNROLL
                idx0 = jnp.broadcast_to((base * N_HEADS).astype(jnp.int32),
                                        (16,))
                for u in range(_UNROLL):
                    for h in range(N_HEADS):
                        wv = plsc.load_gather(wb, [idx0 + (u * N_HEADS + h)])
                        sl = pl.ds(h * HEAD_DIM, HEAD_DIM)
                        mb[base + u, sl] = mb[base + u, sl] * wv
                return carry2

            lax.fori_loop(0, CHUNK // _UNROLL, e_body, 0)

            fire_scatter(k, p)
        return carry

    lax.fori_loop(0, NCHUNKS // NBUF, group_body, 0)
    for p in range(NBUF):
        wait_scatter(NCHUNKS - NBUF + p, p)
    plsc.subcore_barrier()

    rows = pl.ds(s * ROWS_PER_TILE, ROWS_PER_TILE)
    pltpu.sync_copy(acc.at[rows], out_hbm.at[c, rows])


@functools.partial(
    pl.kernel,
    out_type=jax.ShapeDtypeStruct((NUM_CORES, N_PAD, ROW), jnp.float32),
    mesh=plsc.VectorSubcoreMesh(core_axis_name="c", subcore_axis_name="s",
                                num_cores=NUM_CORES, num_subcores=NUM_SUBCORES),
    compiler_params=pltpu.CompilerParams(needs_layout_passes=False,
                                         use_tc_tiling_on_sc=False),
    scratch_types=[
        pltpu.VMEM_SHARED((N_PAD, ROW), jnp.float32),
        pltpu.VMEM((NCHUNKS, CHUNK), jnp.int32),
        pltpu.VMEM((NCHUNKS, CHUNK), jnp.int32),
        pltpu.VMEM((NBUF, CHUNK * N_HEADS), jnp.float32),
        pltpu.VMEM((NBUF, CHUNK, ROW), jnp.float32),
    ] + [pltpu.SemaphoreType.DMA] * (3 * NBUF),
)
def _sc_scatter(v_hbm, srcp_hbm, dstp_hbm, wp_hbm, zeros_hbm, out_hbm,
                acc, src_v, dst_v, w_v, msg_v, *sems):
    _sc_body(v_hbm, srcp_hbm, dstp_hbm, wp_hbm, zeros_hbm, out_hbm,
             acc, src_v, dst_v, w_v, msg_v, *sems)


def _add_body(p_ref, o_ref):
    o_ref[...] = p_ref[0] + p_ref[1]


def _ilv_body(v_ref, o_ref):
    o_ref[...] = jnp.concatenate([v_ref[h] for h in range(N_HEADS)], axis=-1)


_N_ILV_BLOCKS = 10


def _interleave(v3):
    return pl.pallas_call(
        _ilv_body,
        grid=(_N_ILV_BLOCKS,),
        in_specs=[pl.BlockSpec((N_HEADS, N_NODES // _N_ILV_BLOCKS, HEAD_DIM),
                               lambda i: (0, i, 0))],
        out_specs=pl.BlockSpec((N_NODES // _N_ILV_BLOCKS, ROW),
                               lambda i: (i, 0)),
        out_shape=jax.ShapeDtypeStruct((N_NODES, ROW), jnp.float32),
    )(v3)


_N_ADD_BLOCKS = 16


def _combine(partials):
    return pl.pallas_call(
        _add_body,
        grid=(_N_ADD_BLOCKS,),
        in_specs=[pl.BlockSpec((NUM_CORES, N_PAD // _N_ADD_BLOCKS, ROW),
                               lambda i: (0, i, 0))],
        out_specs=pl.BlockSpec((N_PAD // _N_ADD_BLOCKS, ROW), lambda i: (i, 0)),
        out_shape=jax.ShapeDtypeStruct((N_PAD, ROW), jnp.float32),
    )(partials)


@jax.jit
def kernel(V, edge_index, w):
    B, H, N, d = V.shape
    vr = _interleave(V[0])
    src = edge_index[0].astype(jnp.int32)
    dst = edge_index[1].astype(jnp.int32)
    srcp = src.reshape(NUM_WORKERS, NCHUNKS, CHUNK)
    dstp = dst.reshape(NUM_WORKERS, NCHUNKS, CHUNK)
    zeros = jnp.zeros((ROWS_PER_TILE, ROW), jnp.float32)

    wp = w.reshape(N_EDGES * N_HEADS)
    partials = _sc_scatter(vr, srcp, dstp, wp, zeros)
    combined = _combine(partials)[:N_NODES]
    out = combined.reshape(N, H, d).transpose(1, 0, 2)[None]
    return out


# speedup vs baseline: 1.0264x; 1.0264x over previous
import functools

import jax
import jax.numpy as jnp
from jax import lax
from jax.experimental import pallas as pl
from jax.experimental.pallas import tpu as pltpu
from jax.experimental.pallas import tpu_sc as plsc

N_NODES = 10000
N_EDGES = 320000
N_HEADS = 4
HEAD_DIM = 16
ROW = N_HEADS * HEAD_DIM

NUM_CORES = 2
NUM_SUBCORES = 16
NUM_WORKERS = NUM_CORES * NUM_SUBCORES
CHUNK = 200
NCHUNKS = 50
EDGES_PER_TILE = CHUNK * NCHUNKS
NBUF = 5
_UNROLL = 8
ROWS_PER_TILE = 640
N_PAD = NUM_SUBCORES * ROWS_PER_TILE

_IN_BOUNDS = jax.lax.GatherScatterMode.PROMISE_IN_BOUNDS


def _sc_body(v_hbm, srcp_hbm, dstp_hbm, wp_hbm, zeros_hbm, out_hbm,
             acc, src_v, dst_v, w_v, msg_v, *sems):
    gsems = sems[:NBUF]
    ssems = sems[NBUF:2 * NBUF]
    wsems = sems[2 * NBUF:]
    c = lax.axis_index("c")
    s = lax.axis_index("s")
    wid = c * NUM_SUBCORES + s

    pltpu.sync_copy(zeros_hbm, acc.at[pl.ds(s * ROWS_PER_TILE, ROWS_PER_TILE)])
    pltpu.sync_copy(srcp_hbm.at[wid], src_v)
    pltpu.sync_copy(dstp_hbm.at[wid], dst_v)
    plsc.subcore_barrier()

    wbase0 = wid * (EDGES_PER_TILE * N_HEADS)

    def fire_gather(k, b):
        pltpu.async_copy(v_hbm.at[src_v.at[k]], msg_v.at[b], gsems[b])
        pltpu.async_copy(wp_hbm.at[pl.ds(wbase0 + k * (CHUNK * N_HEADS),
                                         CHUNK * N_HEADS)],
                         w_v.at[b], wsems[b])

    def wait_gather(k, b):
        pltpu.make_async_copy(v_hbm.at[src_v.at[k]], msg_v.at[b], gsems[b]).wait()
        pltpu.make_async_copy(wp_hbm.at[pl.ds(wbase0 + k * (CHUNK * N_HEADS),
                                              CHUNK * N_HEADS)],
                              w_v.at[b], wsems[b]).wait()

    def fire_scatter(k, b):
        pltpu.async_copy(msg_v.at[b], acc.at[dst_v.at[k]], ssems[b], add=True)

    def wait_scatter(k, b):
        pltpu.make_async_copy(msg_v.at[b], acc.at[dst_v.at[k]], ssems[b]).wait()

    fire_gather(0, 0)
    fire_gather(1, 1)

    def group_body(g, carry):
        for p in range(NBUF):
            k = g * NBUF + p
            b2 = (p + 2) % NBUF
            @pl.when(k <= NCHUNKS - 3)
            def _():
                @pl.when(k >= 3)
                def _():
                    wait_scatter(k - 3, b2)
                fire_gather(k + 2, b2)

            wait_gather(k, p)
            mb = msg_v.at[p]
            wb = w_v.at[p]

            def e_body(eb, carry2):
                base = eb * _UNROLL
                idx0 = jnp.broadcast_to((base * N_HEADS).astype(jnp.int32),
                                        (16,))
                for u in range(_UNROLL):
                    for h in range(N_HEADS):
                        wv = plsc.load_gather(wb, [idx0 + (u * N_HEADS + h)])
                        sl = pl.ds(h * HEAD_DIM, HEAD_DIM)
                        mb[base + u, sl] = mb[base + u, sl] * wv
                return carry2

            lax.fori_loop(0, CHUNK // _UNROLL, e_body, 0)

            fire_scatter(k, p)
        return carry

    lax.fori_loop(0, NCHUNKS // NBUF, group_body, 0)
    for p in range(NBUF):
        wait_scatter(NCHUNKS - NBUF + p, p)
    plsc.subcore_barrier()

    rows = pl.ds(s * ROWS_PER_TILE, ROWS_PER_TILE)
    pltpu.sync_copy(acc.at[rows], out_hbm.at[c, rows])


@functools.partial(
    pl.kernel,
    out_type=jax.ShapeDtypeStruct((NUM_CORES, N_PAD, ROW), jnp.float32),
    mesh=plsc.VectorSubcoreMesh(core_axis_name="c", subcore_axis_name="s",
                                num_cores=NUM_CORES, num_subcores=NUM_SUBCORES),
    compiler_params=pltpu.CompilerParams(needs_layout_passes=False,
                                         use_tc_tiling_on_sc=False),
    scratch_types=[
        pltpu.VMEM_SHARED((N_PAD, ROW), jnp.float32),
        pltpu.VMEM((NCHUNKS, CHUNK), jnp.int32),
        pltpu.VMEM((NCHUNKS, CHUNK), jnp.int32),
        pltpu.VMEM((NBUF, CHUNK * N_HEADS), jnp.float32),
        pltpu.VMEM((NBUF, CHUNK, ROW), jnp.float32),
    ] + [pltpu.SemaphoreType.DMA] * (3 * NBUF),
)
def _sc_scatter(v_hbm, srcp_hbm, dstp_hbm, wp_hbm, zeros_hbm, out_hbm,
                acc, src_v, dst_v, w_v, msg_v, *sems):
    _sc_body(v_hbm, srcp_hbm, dstp_hbm, wp_hbm, zeros_hbm, out_hbm,
             acc, src_v, dst_v, w_v, msg_v, *sems)


def _add_body(p_ref, o_ref):
    x = p_ref[0] + p_ref[1]
    for h in range(N_HEADS):
        o_ref[h] = x[:, h * HEAD_DIM:(h + 1) * HEAD_DIM]


_N_ADD_BLOCKS = 10


def _combine(partials):
    return pl.pallas_call(
        _add_body,
        grid=(_N_ADD_BLOCKS,),
        in_specs=[pl.BlockSpec((NUM_CORES, N_NODES // _N_ADD_BLOCKS, ROW),
                               lambda i: (0, i, 0))],
        out_specs=pl.BlockSpec((N_HEADS, N_NODES // _N_ADD_BLOCKS, HEAD_DIM),
                               lambda i: (0, i, 0)),
        out_shape=jax.ShapeDtypeStruct((N_HEADS, N_NODES, HEAD_DIM),
                                       jnp.float32),
    )(partials)


@jax.jit
def kernel(V, edge_index, w):
    B, H, N, d = V.shape
    vr = jnp.transpose(V[0], (1, 0, 2)).reshape(N, H * d)
    src = edge_index[0].astype(jnp.int32)
    dst = edge_index[1].astype(jnp.int32)
    srcp = src.reshape(NUM_WORKERS, NCHUNKS, CHUNK)
    dstp = dst.reshape(NUM_WORKERS, NCHUNKS, CHUNK)
    zeros = jnp.zeros((ROWS_PER_TILE, ROW), jnp.float32)

    wp = w.reshape(N_EDGES * N_HEADS)
    partials = _sc_scatter(vr, srcp, dstp, wp, zeros)
    return _combine(partials)[None]


# speedup vs baseline: 1.0447x; 1.0179x over previous
import functools

import jax
import jax.numpy as jnp
from jax import lax
from jax.experimental import pallas as pl
from jax.experimental.pallas import tpu as pltpu
from jax.experimental.pallas import tpu_sc as plsc

N_NODES = 10000
N_EDGES = 320000
N_HEADS = 4
HEAD_DIM = 16
ROW = N_HEADS * HEAD_DIM

NUM_CORES = 2
NUM_SUBCORES = 16
NUM_WORKERS = NUM_CORES * NUM_SUBCORES
CHUNK = 200
NCHUNKS = 50
EDGES_PER_TILE = CHUNK * NCHUNKS
NBUF = 5
_UNROLL = 8
ROWS_PER_TILE = 640
N_PAD = NUM_SUBCORES * ROWS_PER_TILE

_IN_BOUNDS = jax.lax.GatherScatterMode.PROMISE_IN_BOUNDS


def _sc_body(v_hbm, srcp_hbm, dstp_hbm, wp_hbm, zeros_hbm, out_hbm,
             acc, src_v, dst_v, w_v, msg_v, *sems):
    gsems = sems[:NBUF]
    ssems = sems[NBUF:2 * NBUF]
    wsems = sems[2 * NBUF:]
    c = lax.axis_index("c")
    s = lax.axis_index("s")
    wid = c * NUM_SUBCORES + s

    pltpu.sync_copy(zeros_hbm, acc.at[pl.ds(s * ROWS_PER_TILE, ROWS_PER_TILE)])
    pltpu.sync_copy(srcp_hbm.at[wid], src_v)
    pltpu.sync_copy(dstp_hbm.at[wid], dst_v)
    plsc.subcore_barrier()

    wbase0 = wid * (EDGES_PER_TILE * N_HEADS)

    def fire_gather(k, b):
        pltpu.async_copy(v_hbm.at[src_v.at[k]], msg_v.at[b], gsems[b])
        pltpu.async_copy(wp_hbm.at[pl.ds(wbase0 + k * (CHUNK * N_HEADS),
                                         CHUNK * N_HEADS)],
                         w_v.at[b], wsems[b])

    def wait_gather(k, b):
        pltpu.make_async_copy(v_hbm.at[src_v.at[k]], msg_v.at[b], gsems[b]).wait()
        pltpu.make_async_copy(wp_hbm.at[pl.ds(wbase0 + k * (CHUNK * N_HEADS),
                                              CHUNK * N_HEADS)],
                              w_v.at[b], wsems[b]).wait()

    def fire_scatter(k, b):
        pltpu.async_copy(msg_v.at[b], acc.at[dst_v.at[k]], ssems[b], add=True)

    def wait_scatter(k, b):
        pltpu.make_async_copy(msg_v.at[b], acc.at[dst_v.at[k]], ssems[b]).wait()

    fire_gather(0, 0)
    fire_gather(1, 1)

    def group_body(g, carry):
        for p in range(NBUF):
            k = g * NBUF + p
            b2 = (p + 2) % NBUF
            @pl.when(k <= NCHUNKS - 3)
            def _():
                @pl.when(k >= 3)
                def _():
                    wait_scatter(k - 3, b2)
                fire_gather(k + 2, b2)

            wait_gather(k, p)
            mb = msg_v.at[p]
            wb = w_v.at[p]

            def e_body(eb, carry2):
                base = eb * _UNROLL
                idx0 = jnp.broadcast_to((base * N_HEADS).astype(jnp.int32),
                                        (16,))
                for u in range(_UNROLL):
                    for h in range(N_HEADS):
                        wv = plsc.load_gather(wb, [idx0 + (u * N_HEADS + h)])
                        sl = pl.ds(h * HEAD_DIM, HEAD_DIM)
                        mb[base + u, sl] = mb[base + u, sl] * wv
                return carry2

            lax.fori_loop(0, CHUNK // _UNROLL, e_body, 0)

            fire_scatter(k, p)
        return carry

    lax.fori_loop(0, NCHUNKS // NBUF, group_body, 0)
    for p in range(NBUF):
        wait_scatter(NCHUNKS - NBUF + p, p)
    plsc.subcore_barrier()

    rows = pl.ds(s * ROWS_PER_TILE, ROWS_PER_TILE)
    pltpu.sync_copy(acc.at[rows], out_hbm.at[c, rows])


@functools.partial(
    pl.kernel,
    out_type=jax.ShapeDtypeStruct((NUM_CORES, N_PAD, ROW), jnp.float32),
    mesh=plsc.VectorSubcoreMesh(core_axis_name="c", subcore_axis_name="s",
                                num_cores=NUM_CORES, num_subcores=NUM_SUBCORES),
    compiler_params=pltpu.CompilerParams(needs_layout_passes=False,
                                         use_tc_tiling_on_sc=False),
    scratch_types=[
        pltpu.VMEM_SHARED((N_PAD, ROW), jnp.float32),
        pltpu.VMEM((NCHUNKS, CHUNK), jnp.int32),
        pltpu.VMEM((NCHUNKS, CHUNK), jnp.int32),
        pltpu.VMEM((NBUF, CHUNK * N_HEADS), jnp.float32),
        pltpu.VMEM((NBUF, CHUNK, ROW), jnp.float32),
    ] + [pltpu.SemaphoreType.DMA] * (3 * NBUF),
)
def _sc_scatter(v_hbm, srcp_hbm, dstp_hbm, wp_hbm, zeros_hbm, out_hbm,
                acc, src_v, dst_v, w_v, msg_v, *sems):
    _sc_body(v_hbm, srcp_hbm, dstp_hbm, wp_hbm, zeros_hbm, out_hbm,
             acc, src_v, dst_v, w_v, msg_v, *sems)


def _add_body(p_ref, o_ref):
    o_ref[...] = p_ref[0] + p_ref[1]


_N_ADD_BLOCKS = 16


def _combine(partials):
    return pl.pallas_call(
        _add_body,
        grid=(_N_ADD_BLOCKS,),
        in_specs=[pl.BlockSpec((NUM_CORES, N_PAD // _N_ADD_BLOCKS, ROW),
                               lambda i: (0, i, 0))],
        out_specs=pl.BlockSpec((N_PAD // _N_ADD_BLOCKS, ROW), lambda i: (i, 0)),
        out_shape=jax.ShapeDtypeStruct((N_PAD, ROW), jnp.float32),
    )(partials)


@jax.jit
def kernel(V, edge_index, w):
    B, H, N, d = V.shape
    vr = jnp.transpose(V[0], (1, 0, 2)).reshape(N, H * d)
    src = edge_index[0].astype(jnp.int32)
    dst = edge_index[1].astype(jnp.int32)
    srcp = src.reshape(NUM_WORKERS, NCHUNKS, CHUNK)
    dstp = dst.reshape(NUM_WORKERS, NCHUNKS, CHUNK)
    zeros = jnp.zeros((ROWS_PER_TILE, ROW), jnp.float32)

    wp = w.reshape(N_EDGES * N_HEADS)
    partials = _sc_scatter(vr, srcp, dstp, wp, zeros)
    combined = _combine(partials)[:N_NODES]
    out = combined.reshape(N, H, d).transpose(1, 0, 2)[None]
    return out
